# baseline (speedup 1.0000x reference)
import jax
import jax.numpy as jnp
from jax.experimental import pallas as pl

N = 100000
E = 1600000
C = 32
GRID = 4
IMG = 128.0


def _bfdot(a, b):
    return jax.lax.dot(a.astype(jnp.bfloat16), b.astype(jnp.bfloat16),
                       preferred_element_type=jnp.float32)


def kernel(x, pos, edge_index, batch, lin_W, lin_src_W, lin_dst_W, pos_W1, pos_b1, pos_W2, pos_b2, attn_W1, attn_b1, attn_W2, attn_b2, bn_gamma, bn_beta, fc_W):
    src = edge_index[0]
    dst = edge_index[1]
    f32 = jnp.float32

    a_src_n = x * lin_src_W[0]
    a_dst_n = x * lin_dst_W[0]
    v_n = x * lin_W[0]

    pd = pos[dst] - pos[src]
    delta = _bfdot(_bfdot(pd, pos_W1) + pos_b1, pos_W2) + pos_b2
    alpha_pre = a_dst_n[dst] - a_src_n[src] + delta
    alpha = _bfdot(_bfdot(alpha_pre, attn_W1) + attn_b1, attn_W2) + attn_b2

    hi = jax.lax.Precision.HIGHEST
    Wp = jnp.dot(pos_W1, pos_W2, precision=hi)
    bp = pos_b1 @ pos_W2 + pos_b2
    Wa = jnp.dot(attn_W1, attn_W2, precision=hi)
    ba = attn_b1 @ attn_W2 + attn_b2
    PA = jnp.dot(Wp, Wa, precision=hi)
    A_d = x * (lin_dst_W[0] @ Wa) + jnp.dot(pos, PA, precision=hi)
    A_s = -x * (lin_src_W[0] @ Wa) - jnp.dot(pos, PA, precision=hi)
    b_tot = bp @ Wa + ba

    smax = jax.ops.segment_max(A_s[src], dst, num_segments=N)
    m = jnp.where(jnp.isfinite(smax), A_d + smax + b_tot, 0.0)

    e = jnp.exp(alpha - m[dst])
    vd = v_n[src] + delta
    T1 = jax.ops.segment_sum(e, dst, num_segments=N)
    T2 = jax.ops.segment_sum(e * vd, dst, num_segments=N)
    h = T2 / (T1 + 1e-16)

    h = jax.nn.elu(h)
    mean = jnp.mean(h, axis=0)
    var = jnp.var(h, axis=0)
    h = bn_gamma * (h - mean) / jnp.sqrt(var + 1e-5) + bn_beta

    cell = IMG / GRID
    cx = jnp.clip((pos[:, 0] / cell).astype(jnp.int32), 0, GRID - 1)
    cy = jnp.clip((pos[:, 1] / cell).astype(jnp.int32), 0, GRID - 1)
    cluster = batch * (GRID * GRID) + cx * GRID + cy
    pooled = jax.ops.segment_max(h, cluster, num_segments=GRID * GRID)
    pooled = jnp.where(jnp.isfinite(pooled), pooled, 0.0)
    feat = pooled.reshape(-1, GRID * GRID * C)
    return feat @ fc_W


# speedup vs baseline: 1.3524x; 1.3524x over previous
import functools
import jax
import jax.numpy as jnp
from jax.experimental import pallas as pl

N = 100000
E = 1600000
C = 32
GRID = 4
IMG = 128.0
BLK = 12800


def _bf(a):
    return a.astype(jnp.bfloat16)


def _edge_chain_body(g_ref, w1_ref, w2_ref, wa1_ref, wa2_ref, b1_ref, b2_ref,
                     ba1_ref, ba2_ref, wv_ref, ws_ref, wd_ref,
                     alpha_ref, vd_ref):
    g = g_ref[...]
    pd = g[4:7, :] - g[0:3, :]
    w1t = w1_ref[...].T
    t1 = jax.lax.dot(_bf(w1t), _bf(pd), preferred_element_type=jnp.float32)
    t1 = t1 + b1_ref[...].T
    w2t = w2_ref[...].T
    d = jax.lax.dot(_bf(w2t), _bf(t1), preferred_element_type=jnp.float32)
    d = d + b2_ref[...].T
    xs = g[3:4, :]
    xd = g[7:8, :]
    ap = wd_ref[...].T * xd - ws_ref[...].T * xs + d
    wa1t = wa1_ref[...].T
    t2 = jax.lax.dot(_bf(wa1t), _bf(ap), preferred_element_type=jnp.float32)
    t2 = t2 + ba1_ref[...].T
    wa2t = wa2_ref[...].T
    al = jax.lax.dot(_bf(wa2t), _bf(t2), preferred_element_type=jnp.float32)
    al = al + ba2_ref[...].T
    vd = wv_ref[...].T * xs + d
    alpha_ref[...] = al.T
    vd_ref[...] = vd.T


def _edge_chain(G, pos_W1, pos_b1, pos_W2, pos_b2, attn_W1, attn_b1,
                attn_W2, attn_b2, wv, ws, wd):
    grid = E // BLK
    full = lambda shape: pl.BlockSpec(shape, lambda i: (0, 0))
    return pl.pallas_call(
        _edge_chain_body,
        grid=(grid,),
        in_specs=[
            pl.BlockSpec((8, BLK), lambda i: (0, i)),
            full((3, 64)), full((64, 32)), full((32, 64)), full((64, 32)),
            full((1, 64)), full((1, 32)), full((1, 64)), full((1, 32)),
            full((1, 32)), full((1, 32)), full((1, 32)),
        ],
        out_specs=[
            pl.BlockSpec((BLK, 32), lambda i: (i, 0)),
            pl.BlockSpec((BLK, 32), lambda i: (i, 0)),
        ],
        out_shape=[
            jax.ShapeDtypeStruct((E, 32), jnp.float32),
            jax.ShapeDtypeStruct((E, 32), jnp.float32),
        ],
    )(G, pos_W1, pos_W2, attn_W1, attn_W2,
      pos_b1[None, :], pos_b2[None, :], attn_b1[None, :], attn_b2[None, :],
      wv[None, :], ws[None, :], wd[None, :])


def kernel(x, pos, edge_index, batch, lin_W, lin_src_W, lin_dst_W, pos_W1, pos_b1, pos_W2, pos_b2, attn_W1, attn_b1, attn_W2, attn_b2, bn_gamma, bn_beta, fc_W):
    src = edge_index[0]
    dst = edge_index[1]

    tbl = jnp.concatenate([pos, x], axis=1).T
    G = jnp.concatenate([tbl[:, src], tbl[:, dst]], axis=0)

    alpha, vdelta = _edge_chain(G, pos_W1, pos_b1, pos_W2, pos_b2,
                                attn_W1, attn_b1, attn_W2, attn_b2,
                                lin_W[0], lin_src_W[0], lin_dst_W[0])

    hi = jax.lax.Precision.HIGHEST
    Wp = jnp.dot(pos_W1, pos_W2, precision=hi)
    bp = pos_b1 @ pos_W2 + pos_b2
    Wa = jnp.dot(attn_W1, attn_W2, precision=hi)
    ba = attn_b1 @ attn_W2 + attn_b2
    PA = jnp.dot(Wp, Wa, precision=hi)
    A_d = x * (lin_dst_W[0] @ Wa) + jnp.dot(pos, PA, precision=hi)
    A_s = -x * (lin_src_W[0] @ Wa) - jnp.dot(pos, PA, precision=hi)
    b_tot = bp @ Wa + ba

    smax = jax.ops.segment_max(A_s[src], dst, num_segments=N)
    m = jnp.where(jnp.isfinite(smax), A_d + smax + b_tot, 0.0)

    e = jnp.exp(alpha - m[dst])
    T1 = jax.ops.segment_sum(e, dst, num_segments=N)
    T2 = jax.ops.segment_sum(e * vdelta, dst, num_segments=N)
    h = T2 / (T1 + 1e-16)

    h = jax.nn.elu(h)
    mean = jnp.mean(h, axis=0)
    var = jnp.var(h, axis=0)
    h = bn_gamma * (h - mean) / jnp.sqrt(var + 1e-5) + bn_beta

    cell = IMG / GRID
    cx = jnp.clip((pos[:, 0] / cell).astype(jnp.int32), 0, GRID - 1)
    cy = jnp.clip((pos[:, 1] / cell).astype(jnp.int32), 0, GRID - 1)
    cluster = batch * (GRID * GRID) + cx * GRID + cy
    pooled = jax.ops.segment_max(h, cluster, num_segments=GRID * GRID)
    pooled = jnp.where(jnp.isfinite(pooled), pooled, 0.0)
    feat = pooled.reshape(-1, GRID * GRID * C)
    return feat @ fc_W


# speedup vs baseline: 1.4803x; 1.0946x over previous
import functools
import jax
import jax.numpy as jnp
from jax.experimental import pallas as pl

N = 100000
E = 1600000
C = 32
GRID = 4
IMG = 128.0
BLK = 12800


def _bf(a):
    return a.astype(jnp.bfloat16)


def _edge_chain_body(g_ref, w1_ref, w2_ref, wa1_ref, wa2_ref, b1_ref, b2_ref,
                     ba1_ref, ba2_ref, wv_ref, ws_ref, wd_ref,
                     alpha_ref, vd_ref):
    g = g_ref[...]
    pd = g[4:7, :] - g[0:3, :]
    w1t = w1_ref[...].T
    t1 = jax.lax.dot(_bf(w1t), _bf(pd), preferred_element_type=jnp.float32)
    t1 = t1 + b1_ref[...].T
    w2t = w2_ref[...].T
    d = jax.lax.dot(_bf(w2t), _bf(t1), preferred_element_type=jnp.float32)
    d = d + b2_ref[...].T
    xs = g[3:4, :]
    xd = g[7:8, :]
    ap = wd_ref[...].T * xd - ws_ref[...].T * xs + d
    wa1t = wa1_ref[...].T
    t2 = jax.lax.dot(_bf(wa1t), _bf(ap), preferred_element_type=jnp.float32)
    t2 = t2 + ba1_ref[...].T
    wa2t = wa2_ref[...].T
    al = jax.lax.dot(_bf(wa2t), _bf(t2), preferred_element_type=jnp.float32)
    al = al + ba2_ref[...].T
    vd = wv_ref[...].T * xs + d
    alpha_ref[...] = al.T
    vd_ref[...] = vd.T


def _edge_chain(G, pos_W1, pos_b1, pos_W2, pos_b2, attn_W1, attn_b1,
                attn_W2, attn_b2, wv, ws, wd):
    grid = E // BLK
    full = lambda shape: pl.BlockSpec(shape, lambda i: (0, 0))
    return pl.pallas_call(
        _edge_chain_body,
        grid=(grid,),
        in_specs=[
            pl.BlockSpec((8, BLK), lambda i: (0, i)),
            full((3, 64)), full((64, 32)), full((32, 64)), full((64, 32)),
            full((1, 64)), full((1, 32)), full((1, 64)), full((1, 32)),
            full((1, 32)), full((1, 32)), full((1, 32)),
        ],
        out_specs=[
            pl.BlockSpec((BLK, 32), lambda i: (i, 0)),
            pl.BlockSpec((BLK, 32), lambda i: (i, 0)),
        ],
        out_shape=[
            jax.ShapeDtypeStruct((E, 32), jnp.float32),
            jax.ShapeDtypeStruct((E, 32), jnp.float32),
        ],
    )(G, pos_W1, pos_W2, attn_W1, attn_W2,
      pos_b1[None, :], pos_b2[None, :], attn_b1[None, :], attn_b2[None, :],
      wv[None, :], ws[None, :], wd[None, :])


def kernel(x, pos, edge_index, batch, lin_W, lin_src_W, lin_dst_W, pos_W1, pos_b1, pos_W2, pos_b2, attn_W1, attn_b1, attn_W2, attn_b2, bn_gamma, bn_beta, fc_W):
    src = edge_index[0]
    dst = edge_index[1]

    tbl = jnp.concatenate([pos, x], axis=1).T
    G = jnp.concatenate([tbl[:, src], tbl[:, dst]], axis=0)

    alpha, vdelta = _edge_chain(G, pos_W1, pos_b1, pos_W2, pos_b2,
                                attn_W1, attn_b1, attn_W2, attn_b2,
                                lin_W[0], lin_src_W[0], lin_dst_W[0])

    hi = jax.lax.Precision.HIGHEST
    Wp = jnp.dot(pos_W1, pos_W2, precision=hi)
    bp = pos_b1 @ pos_W2 + pos_b2
    Wa = jnp.dot(attn_W1, attn_W2, precision=hi)
    ba = attn_b1 @ attn_W2 + attn_b2
    PA = jnp.dot(Wp, Wa, precision=hi)
    A_d = x * (lin_dst_W[0] @ Wa) + jnp.dot(pos, PA, precision=hi)
    A_s = -x * (lin_src_W[0] @ Wa) - jnp.dot(pos, PA, precision=hi)
    b_tot = bp @ Wa + ba

    smax = jax.ops.segment_max(A_s[src], dst, num_segments=N)
    m = jnp.where(jnp.isfinite(smax), A_d + smax + b_tot, 0.0)

    e = jnp.exp(alpha - m[dst])
    T = jax.ops.segment_sum(jnp.concatenate([e, e * vdelta], axis=1),
                            dst, num_segments=N)
    h = T[:, 32:] / (T[:, :32] + 1e-16)

    h = jax.nn.elu(h)
    mean = jnp.mean(h, axis=0)
    var = jnp.var(h, axis=0)
    h = bn_gamma * (h - mean) / jnp.sqrt(var + 1e-5) + bn_beta

    cell = IMG / GRID
    cx = jnp.clip((pos[:, 0] / cell).astype(jnp.int32), 0, GRID - 1)
    cy = jnp.clip((pos[:, 1] / cell).astype(jnp.int32), 0, GRID - 1)
    cluster = batch * (GRID * GRID) + cx * GRID + cy
    pooled = jax.ops.segment_max(h, cluster, num_segments=GRID * GRID)
    pooled = jnp.where(jnp.isfinite(pooled), pooled, 0.0)
    feat = pooled.reshape(-1, GRID * GRID * C)
    return feat @ fc_W


# speedup vs baseline: 1.7147x; 1.1584x over previous
import functools
import jax
import jax.numpy as jnp
from jax.experimental import pallas as pl

N = 100000
E = 1600000
C = 32
GRID = 4
IMG = 128.0
BLK = 12800


def _bf(a):
    return a.astype(jnp.bfloat16)


def _edge_chain_body(g_ref, w1_ref, w2_ref, wa1_ref, wa2_ref, b1_ref, b2_ref,
                     ba1_ref, ba2_ref, wv_ref, ws_ref, wd_ref, pa_ref, us_ref,
                     alpha_ref, vd_ref, asrc_ref):
    g = g_ref[...]
    pd = g[4:7, :] - g[0:3, :]
    w1t = w1_ref[...].T
    t1 = jax.lax.dot(_bf(w1t), _bf(pd), preferred_element_type=jnp.float32)
    t1 = t1 + b1_ref[...].T
    w2t = w2_ref[...].T
    d = jax.lax.dot(_bf(w2t), _bf(t1), preferred_element_type=jnp.float32)
    d = d + b2_ref[...].T
    xs = g[3:4, :]
    xd = g[7:8, :]
    ap = wd_ref[...].T * xd - ws_ref[...].T * xs + d
    wa1t = wa1_ref[...].T
    t2 = jax.lax.dot(_bf(wa1t), _bf(ap), preferred_element_type=jnp.float32)
    t2 = t2 + ba1_ref[...].T
    wa2t = wa2_ref[...].T
    al = jax.lax.dot(_bf(wa2t), _bf(t2), preferred_element_type=jnp.float32)
    al = al + ba2_ref[...].T
    vd = wv_ref[...].T * xs + d
    asrc = -(us_ref[...].T * xs +
             jax.lax.dot(pa_ref[...].T, g[0:3, :],
                         preferred_element_type=jnp.float32))
    alpha_ref[...] = al.T
    vd_ref[...] = vd.T
    asrc_ref[...] = asrc.T


def _edge_chain(G, pos_W1, pos_b1, pos_W2, pos_b2, attn_W1, attn_b1,
                attn_W2, attn_b2, wv, ws, wd, PA, us):
    grid = E // BLK
    full = lambda shape: pl.BlockSpec(shape, lambda i: (0, 0))
    return pl.pallas_call(
        _edge_chain_body,
        grid=(grid,),
        in_specs=[
            pl.BlockSpec((8, BLK), lambda i: (0, i)),
            full((3, 64)), full((64, 32)), full((32, 64)), full((64, 32)),
            full((1, 64)), full((1, 32)), full((1, 64)), full((1, 32)),
            full((1, 32)), full((1, 32)), full((1, 32)),
            full((3, 32)), full((1, 32)),
        ],
        out_specs=[
            pl.BlockSpec((BLK, 32), lambda i: (i, 0)),
            pl.BlockSpec((BLK, 32), lambda i: (i, 0)),
            pl.BlockSpec((BLK, 32), lambda i: (i, 0)),
        ],
        out_shape=[
            jax.ShapeDtypeStruct((E, 32), jnp.float32),
            jax.ShapeDtypeStruct((E, 32), jnp.float32),
            jax.ShapeDtypeStruct((E, 32), jnp.float32),
        ],
    )(G, pos_W1, pos_W2, attn_W1, attn_W2,
      pos_b1[None, :], pos_b2[None, :], attn_b1[None, :], attn_b2[None, :],
      wv[None, :], ws[None, :], wd[None, :], PA, us[None, :])


def kernel(x, pos, edge_index, batch, lin_W, lin_src_W, lin_dst_W, pos_W1, pos_b1, pos_W2, pos_b2, attn_W1, attn_b1, attn_W2, attn_b2, bn_gamma, bn_beta, fc_W):
    src = edge_index[0]
    dst = edge_index[1]

    tbl = jnp.concatenate([pos, x], axis=1).T
    G = jnp.concatenate([tbl[:, src], tbl[:, dst]], axis=0)

    hi = jax.lax.Precision.HIGHEST
    Wp = jnp.dot(pos_W1, pos_W2, precision=hi)
    bp = pos_b1 @ pos_W2 + pos_b2
    Wa = jnp.dot(attn_W1, attn_W2, precision=hi)
    ba = attn_b1 @ attn_W2 + attn_b2
    PA = jnp.dot(Wp, Wa, precision=hi)
    A_d = x * (lin_dst_W[0] @ Wa) + jnp.dot(pos, PA, precision=hi)
    b_tot = bp @ Wa + ba

    alpha, vdelta, asrc = _edge_chain(G, pos_W1, pos_b1, pos_W2, pos_b2,
                                      attn_W1, attn_b1, attn_W2, attn_b2,
                                      lin_W[0], lin_src_W[0], lin_dst_W[0],
                                      PA, lin_src_W[0] @ Wa)

    smax = jax.ops.segment_max(asrc, dst, num_segments=N)
    m = jnp.where(jnp.isfinite(smax), A_d + smax + b_tot, 0.0)

    e = jnp.exp(alpha - m[dst])
    T = jax.ops.segment_sum(jnp.concatenate([e, e * vdelta], axis=1),
                            dst, num_segments=N)
    h = T[:, 32:] / (T[:, :32] + 1e-16)

    h = jax.nn.elu(h)
    mean = jnp.mean(h, axis=0)
    var = jnp.var(h, axis=0)
    h = bn_gamma * (h - mean) / jnp.sqrt(var + 1e-5) + bn_beta

    cell = IMG / GRID
    cx = jnp.clip((pos[:, 0] / cell).astype(jnp.int32), 0, GRID - 1)
    cy = jnp.clip((pos[:, 1] / cell).astype(jnp.int32), 0, GRID - 1)
    cluster = batch * (GRID * GRID) + cx * GRID + cy
    pooled = jax.ops.segment_max(h, cluster, num_segments=GRID * GRID)
    pooled = jnp.where(jnp.isfinite(pooled), pooled, 0.0)
    feat = pooled.reshape(-1, GRID * GRID * C)
    return feat @ fc_W
